# baseline (speedup 1.0000x reference)
import functools

import jax
import jax.numpy as jnp
from jax import lax
from jax.experimental import pallas as pl
from jax.experimental.pallas import tpu as pltpu
from jax.experimental.pallas import tpu_sc as plsc

_B, _V, _C, _Q = 2, 6, 256, 2048
_H0, _W0 = 56, 100
_H1, _W1 = 28, 50
_HW0, _HW1 = _H0 * _W0, _H1 * _W1
_NC, _NS = 2, 16
_NW = _NC * _NS
_CPT = _C // _NW
_QCH = 256
_NQC = _Q // _QCH
_R = 2.0
_KP = ((0.0, 0.0, 0.0), (_R, 0.0, 0.0), (0.0, _R, 0.0), (-_R, 0.0, 0.0))


def _proj_body(refs_ref, intr_ref, extr_ref, idx_ref, w_ref):
    for b in range(_B):
        px = refs_ref[b, 0]
        py = refs_ref[b, 1]
        pz = refs_ref[b, 2]
        for l, (H, W) in enumerate(((_H0, _W0), (_H1, _W1))):
            cnt = jnp.zeros((16, 128), jnp.float32)
            for v in range(_V):
                def _r(x):
                    return x.astype(jnp.bfloat16).astype(jnp.float32)
                e = [[_r(extr_ref[b, v, i, j]) for j in range(4)]
                     for i in range(3)]
                k = [[_r(intr_ref[b, v, i, j]) for j in range(3)]
                     for i in range(3)]
                for kpi, (ox, oy, oz) in enumerate(_KP):
                    X = _r(px + ox)
                    Y = _r(py + oy)
                    Z = _r(pz + oz)
                    cx = _r(e[0][0] * X + e[0][1] * Y + e[0][2] * Z + e[0][3])
                    cy = _r(e[1][0] * X + e[1][1] * Y + e[1][2] * Z + e[1][3])
                    cz = _r(e[2][0] * X + e[2][1] * Y + e[2][2] * Z + e[2][3])
                    p0 = k[0][0] * cx + k[0][1] * cy + k[0][2] * cz
                    p1 = k[1][0] * cx + k[1][1] * cy + k[1][2] * cz
                    z = k[2][0] * cx + k[2][1] * cy + k[2][2] * cz
                    denom = jnp.where(jnp.abs(z) > 1e-6, z, 1e-6)
                    u = p0 / denom
                    vv = p1 / denom
                    valid = (z > 0.0).astype(jnp.float32)
                    cnt = cnt + valid
                    gx = 2.0 * u / float(W - 1) - 1.0
                    gy = 2.0 * vv / float(H - 1) - 1.0
                    x = (gx + 1.0) * 0.5 * float(W - 1)
                    y = (gy + 1.0) * 0.5 * float(H - 1)
                    x0 = jnp.floor(x)
                    y0 = jnp.floor(y)
                    x1 = x0 + 1.0
                    y1 = y0 + 1.0
                    wx1 = x - x0
                    wx0 = 1.0 - wx1
                    wy1 = y - y0
                    wy0 = 1.0 - wy1
                    taps = ((x0, y0, wx0 * wy0), (x1, y0, wx1 * wy0),
                            (x0, y1, wx0 * wy1), (x1, y1, wx1 * wy1))
                    for ti, (xi, yi, wt) in enumerate(taps):
                        inb = ((xi >= 0) & (xi <= W - 1)
                               & (yi >= 0) & (yi <= H - 1)).astype(jnp.float32)
                        xc = jnp.clip(xi, 0, W - 1).astype(jnp.int32)
                        yc = jnp.clip(yi, 0, H - 1).astype(jnp.int32)
                        idx_ref[b, l, v, kpi * 4 + ti] = yc * W + xc
                        w_ref[b, l, v, kpi * 4 + ti] = wt * inb * valid
            rcp = 0.5 / jnp.maximum(cnt, 1.0)
            for v in range(_V):
                for t in range(16):
                    w_ref[b, l, v, t] = w_ref[b, l, v, t] * rcp


def _sc_body(f0_hbm, f1_hbm, idx_hbm, w_hbm, out_hbm,
             slab0, slab1, idxb, wb, acc, sem_i, sem_w):
    wid = lax.axis_index("s") * _NC + lax.axis_index("c")
    for b in range(_B):
        def _zero(i, c):
            for ch in range(_CPT):
                acc[ch, pl.ds(i * 16, 16)] = jnp.zeros((16,), jnp.float32)
            return c
        lax.fori_loop(0, _Q // 16, _zero, 0)
        for v in range(_V):
            pltpu.sync_copy(f0_hbm.at[b, v, wid], slab0)
            pltpu.sync_copy(f1_hbm.at[b, v, wid], slab1)
            pltpu.make_async_copy(
                idx_hbm.at[b, v, 0], idxb.at[0], sem_i).start()
            pltpu.make_async_copy(
                w_hbm.at[b, v, 0], wb.at[0], sem_w).start()

            def _qc(qc, c, b=b, v=v):
                buf = lax.rem(qc, 2)
                nbuf = lax.rem(qc + 1, 2)
                pltpu.make_async_copy(
                    idx_hbm.at[b, v, qc], idxb.at[buf], sem_i).wait()
                pltpu.make_async_copy(
                    w_hbm.at[b, v, qc], wb.at[buf], sem_w).wait()

                @pl.when(qc < _NQC - 1)
                def _():
                    pltpu.make_async_copy(
                        idx_hbm.at[b, v, qc + 1], idxb.at[nbuf],
                        sem_i).start()
                    pltpu.make_async_copy(
                        w_hbm.at[b, v, qc + 1], wb.at[nbuf],
                        sem_w).start()

                def _qg(g, c2):
                    base = g * 16
                    accv = tuple(jnp.zeros((16,), jnp.float32)
                                 for _ in range(_CPT))
                    for l, (slab, hw) in enumerate(((slab0, _HW0),
                                                    (slab1, _HW1))):
                        def _tap(t, accs, l=l, slab=slab, hw=hw):
                            for j in range(2):
                                tt = t * 2 + j
                                iv = idxb[buf, l, tt, pl.ds(base, 16)]
                                wv = wb[buf, l, tt, pl.ds(base, 16)]
                                new = list(accs)
                                for k in range(_CPT // 2):
                                    ix = iv if k == 0 else iv + (k * hw)
                                    gw = plsc.load_gather(slab, [ix])
                                    bb = plsc.bitcast(gw, jnp.bfloat16)
                                    lo, hi = plsc.unpack(
                                        bb, format=plsc.PackFormat.INTERLEAVED)
                                    new[2 * k] = new[2 * k] + wv * lo
                                    new[2 * k + 1] = new[2 * k + 1] + wv * hi
                                accs = tuple(new)
                            return accs
                        accv = lax.fori_loop(0, 8, _tap, accv)
                    for ch in range(_CPT):
                        plsc.addupdate(
                            acc.at[ch, pl.ds(qc * _QCH + base, 16)],
                            accv[ch])
                    return c2
                lax.fori_loop(0, _QCH // 16, _qg, 0)
                return c
            lax.fori_loop(0, _NQC, _qc, 0)
        pltpu.sync_copy(acc, out_hbm.at[wid, b])


def _stage1(refs_r, intrinsics, extrinsics, interpret=False):
    return pl.pallas_call(
        _proj_body,
        out_shape=(
            jax.ShapeDtypeStruct((_B, 2, _V, 16, 16, 128), jnp.int32),
            jax.ShapeDtypeStruct((_B, 2, _V, 16, 16, 128), jnp.float32),
        ),
        in_specs=[
            pl.BlockSpec(memory_space=pltpu.VMEM),
            pl.BlockSpec(memory_space=pltpu.SMEM),
            pl.BlockSpec(memory_space=pltpu.SMEM),
        ],
        interpret=interpret,
    )(refs_r, intrinsics, extrinsics)


def _reorder(a):
    a = a.reshape(_B, 2, _V, 16, _NQC, _QCH)
    return a.transpose(0, 2, 4, 1, 3, 5)


def kernel(features_0, features_1, refs_xyz, intrinsics, extrinsics):
    refs_r = refs_xyz.transpose(0, 2, 1).reshape(_B, 3, 16, 128)
    idx6, w6 = _stage1(refs_r, intrinsics, extrinsics)
    idx_sc = _reorder(idx6)
    w_sc = _reorder(w6)

    def _pack_feats(f, hw):
        fb = f.astype(jnp.bfloat16).reshape(_B, _V, _NW, _CPT // 2, 2, hw)
        fb = fb.transpose(0, 1, 2, 3, 5, 4)
        w = lax.bitcast_convert_type(fb, jnp.int32)
        return w.reshape(_B, _V, _NW, (_CPT // 2) * hw)

    f0r = _pack_feats(features_0, _HW0)
    f1r = _pack_feats(features_1, _HW1)
    mesh = plsc.VectorSubcoreMesh(core_axis_name="c", subcore_axis_name="s")
    sc = pl.kernel(
        _sc_body,
        mesh=mesh,
        compiler_params=pltpu.CompilerParams(needs_layout_passes=False),
        out_type=jax.ShapeDtypeStruct((_NW, _B, _CPT, _Q), jnp.float32),
        scratch_types=[
            pltpu.VMEM(((_CPT // 2) * _HW0,), jnp.int32),
            pltpu.VMEM(((_CPT // 2) * _HW1,), jnp.int32),
            pltpu.VMEM((2, 2, 16, _QCH), jnp.int32),
            pltpu.VMEM((2, 2, 16, _QCH), jnp.float32),
            pltpu.VMEM((_CPT, _Q), jnp.float32),
            pltpu.SemaphoreType.DMA,
            pltpu.SemaphoreType.DMA,
        ],
    )
    out = sc(f0r, f1r, idx_sc, w_sc)
    return out.transpose(1, 3, 0, 2).reshape(_B, _Q, _C)


# speedup vs baseline: 1.3781x; 1.3781x over previous
import functools

import jax
import jax.numpy as jnp
from jax import lax
from jax.experimental import pallas as pl
from jax.experimental.pallas import tpu as pltpu
from jax.experimental.pallas import tpu_sc as plsc

_B, _V, _C, _Q = 2, 6, 256, 2048
_H0, _W0 = 56, 100
_H1, _W1 = 28, 50
_HW0, _HW1 = _H0 * _W0, _H1 * _W1
_NC, _NS = 2, 16
_NW = _NC * _NS
_CPT = _C // _NW
_QCH = 256
_NQC = _Q // _QCH
_R = 2.0
_KP = ((0.0, 0.0, 0.0), (_R, 0.0, 0.0), (0.0, _R, 0.0), (-_R, 0.0, 0.0))


def _proj_body(refs_ref, intr_ref, extr_ref, idx_ref, w_ref):
    for b in range(_B):
        px = refs_ref[b, 0]
        py = refs_ref[b, 1]
        pz = refs_ref[b, 2]
        for l, (H, W) in enumerate(((_H0, _W0), (_H1, _W1))):
            cnt = jnp.zeros((16, 128), jnp.float32)
            for v in range(_V):
                def _r(x):
                    return x.astype(jnp.bfloat16).astype(jnp.float32)
                e = [[_r(extr_ref[b, v, i, j]) for j in range(4)]
                     for i in range(3)]
                k = [[_r(intr_ref[b, v, i, j]) for j in range(3)]
                     for i in range(3)]
                for kpi, (ox, oy, oz) in enumerate(_KP):
                    X = _r(px + ox)
                    Y = _r(py + oy)
                    Z = _r(pz + oz)
                    cx = _r(e[0][0] * X + e[0][1] * Y + e[0][2] * Z + e[0][3])
                    cy = _r(e[1][0] * X + e[1][1] * Y + e[1][2] * Z + e[1][3])
                    cz = _r(e[2][0] * X + e[2][1] * Y + e[2][2] * Z + e[2][3])
                    p0 = k[0][0] * cx + k[0][1] * cy + k[0][2] * cz
                    p1 = k[1][0] * cx + k[1][1] * cy + k[1][2] * cz
                    z = k[2][0] * cx + k[2][1] * cy + k[2][2] * cz
                    denom = jnp.where(jnp.abs(z) > 1e-6, z, 1e-6)
                    u = p0 / denom
                    vv = p1 / denom
                    valid = (z > 0.0).astype(jnp.float32)
                    cnt = cnt + valid
                    gx = 2.0 * u / float(W - 1) - 1.0
                    gy = 2.0 * vv / float(H - 1) - 1.0
                    x = (gx + 1.0) * 0.5 * float(W - 1)
                    y = (gy + 1.0) * 0.5 * float(H - 1)
                    x0 = jnp.floor(x)
                    y0 = jnp.floor(y)
                    x1 = x0 + 1.0
                    y1 = y0 + 1.0
                    wx1 = x - x0
                    wx0 = 1.0 - wx1
                    wy1 = y - y0
                    wy0 = 1.0 - wy1
                    taps = ((x0, y0, wx0 * wy0), (x1, y0, wx1 * wy0),
                            (x0, y1, wx0 * wy1), (x1, y1, wx1 * wy1))
                    for ti, (xi, yi, wt) in enumerate(taps):
                        inb = ((xi >= 0) & (xi <= W - 1)
                               & (yi >= 0) & (yi <= H - 1)).astype(jnp.float32)
                        xc = jnp.clip(xi, 0, W - 1).astype(jnp.int32)
                        yc = jnp.clip(yi, 0, H - 1).astype(jnp.int32)
                        idx_ref[b, l, v, kpi * 4 + ti] = yc * W + xc
                        w_ref[b, l, v, kpi * 4 + ti] = wt * inb * valid
            rcp = 0.5 / jnp.maximum(cnt, 1.0)
            for v in range(_V):
                for t in range(16):
                    w_ref[b, l, v, t] = w_ref[b, l, v, t] * rcp


def _pack_body(in_ref, out_ref):
    x = in_ref[0, 0]

    def _rne(v):
        u = lax.bitcast_convert_type(v, jnp.uint32)
        return ((u + jnp.uint32(0x7FFF) + ((u >> 16) & jnp.uint32(1)))
                & jnp.uint32(0xFFFF0000))

    w = (_rne(x[: _C // 2]) >> 16) | _rne(x[_C // 2:])
    out_ref[0, 0] = lax.bitcast_convert_type(w, jnp.int32)


def _pack_call(f, hw):
    f = f.reshape(_B, _V, _C, hw)
    out = pl.pallas_call(
        _pack_body,
        grid=(_B, _V),
        in_specs=[pl.BlockSpec((1, 1, _C, hw), lambda b, v: (b, v, 0, 0))],
        out_specs=pl.BlockSpec((1, 1, _C // 2, hw), lambda b, v: (b, v, 0, 0)),
        out_shape=jax.ShapeDtypeStruct((_B, _V, _C // 2, hw), jnp.int32),
    )(f)
    return out.reshape(_B, _V, _NW, (_CPT // 2) * hw)


def _sc_body(f0_hbm, f1_hbm, idx_hbm, w_hbm, out_hbm,
             slab0, slab1, idxb, wb, acc, sem_i, sem_w):
    wid = lax.axis_index("s") * _NC + lax.axis_index("c")
    for b in range(_B):
        def _zero(i, c):
            for ch in range(_CPT):
                acc[ch, pl.ds(i * 16, 16)] = jnp.zeros((16,), jnp.float32)
            return c
        lax.fori_loop(0, _Q // 16, _zero, 0)

        def _view(v, c, b=b):
            pltpu.sync_copy(f0_hbm.at[b, v, wid], slab0)
            pltpu.sync_copy(f1_hbm.at[b, v, wid], slab1)
            pltpu.make_async_copy(
                idx_hbm.at[b, :, v, :, pl.ds(0, _QCH)],
                idxb.at[0], sem_i).start()
            pltpu.make_async_copy(
                w_hbm.at[b, :, v, :, pl.ds(0, _QCH)],
                wb.at[0], sem_w).start()

            def _qc(qc, c2):
                buf = lax.rem(qc, 2)
                nbuf = lax.rem(qc + 1, 2)
                qoff = qc * _QCH
                pltpu.make_async_copy(
                    idx_hbm.at[b, :, v, :, pl.ds(qoff, _QCH)],
                    idxb.at[buf], sem_i).wait()
                pltpu.make_async_copy(
                    w_hbm.at[b, :, v, :, pl.ds(qoff, _QCH)],
                    wb.at[buf], sem_w).wait()

                @pl.when(qc < _NQC - 1)
                def _():
                    pltpu.make_async_copy(
                        idx_hbm.at[b, :, v, :, pl.ds(qoff + _QCH, _QCH)],
                        idxb.at[nbuf], sem_i).start()
                    pltpu.make_async_copy(
                        w_hbm.at[b, :, v, :, pl.ds(qoff + _QCH, _QCH)],
                        wb.at[nbuf], sem_w).start()

                def _qg(g, c3):
                    base = g * 16
                    accv = [jnp.zeros((16,), jnp.float32)
                            for _ in range(_CPT)]
                    for l, (slab, hw) in enumerate(((slab0, _HW0),
                                                    (slab1, _HW1))):
                        for tt in range(16):
                            iv = idxb[buf, l, tt, pl.ds(base, 16)]
                            wv = wb[buf, l, tt, pl.ds(base, 16)]
                            for k in range(_CPT // 2):
                                ix = iv if k == 0 else iv + (k * hw)
                                gw = plsc.load_gather(slab, [ix])
                                bb = plsc.bitcast(gw, jnp.bfloat16)
                                lo, hi = plsc.unpack(
                                    bb, format=plsc.PackFormat.INTERLEAVED)
                                accv[k] = accv[k] + wv * lo
                                accv[k + 4] = accv[k + 4] + wv * hi
                    for ch in range(_CPT):
                        plsc.addupdate(
                            acc.at[ch, pl.ds(qoff + base, 16)],
                            accv[ch])
                    return c3
                lax.fori_loop(0, _QCH // 16, _qg, 0)
                return c2
            lax.fori_loop(0, _NQC, _qc, 0)
            return c
        lax.fori_loop(0, _V, _view, 0)
        pltpu.sync_copy(acc, out_hbm.at[wid, b])


def _stage1(refs_r, intrinsics, extrinsics, interpret=False):
    return pl.pallas_call(
        _proj_body,
        out_shape=(
            jax.ShapeDtypeStruct((_B, 2, _V, 16, 16, 128), jnp.int32),
            jax.ShapeDtypeStruct((_B, 2, _V, 16, 16, 128), jnp.float32),
        ),
        in_specs=[
            pl.BlockSpec(memory_space=pltpu.VMEM),
            pl.BlockSpec(memory_space=pltpu.SMEM),
            pl.BlockSpec(memory_space=pltpu.SMEM),
        ],
        interpret=interpret,
    )(refs_r, intrinsics, extrinsics)


def kernel(features_0, features_1, refs_xyz, intrinsics, extrinsics):
    refs_r = refs_xyz.transpose(0, 2, 1).reshape(_B, 3, 16, 128)
    idx6, w6 = _stage1(refs_r, intrinsics, extrinsics)
    idx_sc = idx6.reshape(_B, 2, _V, 16, _Q)
    w_sc = w6.reshape(_B, 2, _V, 16, _Q)
    f0r = _pack_call(features_0, _HW0)
    f1r = _pack_call(features_1, _HW1)
    mesh = plsc.VectorSubcoreMesh(core_axis_name="c", subcore_axis_name="s")
    sc = pl.kernel(
        _sc_body,
        mesh=mesh,
        compiler_params=pltpu.CompilerParams(needs_layout_passes=False),
        out_type=jax.ShapeDtypeStruct((_NW, _B, _CPT, _Q), jnp.float32),
        scratch_types=[
            pltpu.VMEM(((_CPT // 2) * _HW0,), jnp.int32),
            pltpu.VMEM(((_CPT // 2) * _HW1,), jnp.int32),
            pltpu.VMEM((2, 2, 16, _QCH), jnp.int32),
            pltpu.VMEM((2, 2, 16, _QCH), jnp.float32),
            pltpu.VMEM((_CPT, _Q), jnp.float32),
            pltpu.SemaphoreType.DMA,
            pltpu.SemaphoreType.DMA,
        ],
    )
    out = sc(f0r, f1r, idx_sc, w_sc)
    o = out.reshape(_NW, _B, 2, _CPT // 2, _Q)
    o = o.transpose(1, 4, 2, 0, 3)
    return o.reshape(_B, _Q, _C)
